# baseline (speedup 1.0000x reference)
import contextlib
import functools

import jax
import jax.numpy as jnp
import numpy as np
from jax import lax
from jax.experimental import pallas as pl
from jax.experimental.pallas import tpu as pltpu
from jax.experimental.pallas import tpu_sc as plsc

B = 16
N = 16384
C = 256
NSAMPLE = 4096

_NC = 2
_NS = 16
_NW = _NC * _NS
_RPW = B * NSAMPLE // _NW
_CHUNK = 64
_NCHUNK = _RPW // _CHUNK


def _gather_body(feat_hbm, xyz_hbm, idxf_hbm,
                 outf_hbm, outx_hbm,
                 idxf_v, outx_v, xyz_v, buf0, buf1, buf2,
                 gsem0, gsem1, gsem2, ssem0, ssem1, ssem2):
    wid = lax.axis_index("s") * _NC + lax.axis_index("c")
    b = wid // 2
    h = wid % 2

    pltpu.sync_copy(idxf_hbm.at[wid], idxf_v)

    gsems = (gsem0, gsem1, gsem2)
    ssems = (ssem0, ssem1, ssem2)
    row_base = wid * _RPW
    bufs = (buf0, buf1, buf2)
    cps = [None, None, None]
    scs = [None, None, None]

    for k in range(2):
        cps[k] = pltpu.async_copy(
            feat_hbm.at[idxf_v.at[k]], bufs[k], gsems[k])

    for d in range(3):
        pltpu.sync_copy(xyz_hbm.at[d, pl.ds(b, 1)],
                        xyz_v.at[pl.ds(d, 1)])

    for i in range(_NCHUNK):
        for k in range(4):
            ids = idxf_v[i, pl.ds(k * 16, 16)] - b * N
            for d in range(3):
                dvec = jnp.full((16,), d, jnp.int32)
                outx_v[d, pl.ds((4 * i + k) * 16, 16)] = plsc.load_gather(
                    xyz_v, [dvec, ids])
        cps[i % 3].wait()
        scs[i % 3] = pltpu.async_copy(
            bufs[i % 3],
            outf_hbm.at[pl.ds(row_base + i * _CHUNK, _CHUNK)],
            ssems[i % 3])
        nxt = i + 2
        if nxt < _NCHUNK:
            if i >= 1:
                scs[(i - 1) % 3].wait()
            cps[nxt % 3] = pltpu.async_copy(
                feat_hbm.at[idxf_v.at[nxt]], bufs[nxt % 3], gsems[nxt % 3])
    for i in range(_NCHUNK - 3, _NCHUNK):
        scs[i % 3].wait()
    for d in range(3):
        pltpu.sync_copy(outx_v.at[pl.ds(d, 1)],
                        outx_hbm.at[d, pl.ds(b, 1),
                                    pl.ds(h * _RPW, _RPW)])


@functools.partial(
    pl.kernel,
    out_type=(
        jax.ShapeDtypeStruct((B * NSAMPLE, C), jnp.float32),
        jax.ShapeDtypeStruct((3, B, NSAMPLE), jnp.float32),
    ),
    mesh=plsc.VectorSubcoreMesh(core_axis_name="c", subcore_axis_name="s"),
    compiler_params=pltpu.CompilerParams(needs_layout_passes=False),
    scratch_types=[
        pltpu.VMEM((_NCHUNK, _CHUNK), jnp.int32),
        pltpu.VMEM((3, _RPW), jnp.float32),
        pltpu.VMEM((3, N), jnp.float32),
        pltpu.VMEM((_CHUNK, C), jnp.float32),
        pltpu.VMEM((_CHUNK, C), jnp.float32),
        pltpu.VMEM((_CHUNK, C), jnp.float32),
        pltpu.SemaphoreType.DMA,
        pltpu.SemaphoreType.DMA,
        pltpu.SemaphoreType.DMA,
        pltpu.SemaphoreType.DMA,
        pltpu.SemaphoreType.DMA,
        pltpu.SemaphoreType.DMA,
    ],
)
def _sc_gather(feat_hbm, xyz_hbm, idxf_hbm, outf_hbm, outx_hbm,
               idxf_v, outx_v, xyz_v, buf0, buf1, buf2,
               gsem0, gsem1, gsem2, ssem0, ssem1, ssem2):
    _gather_body(feat_hbm, xyz_hbm, idxf_hbm,
                 outf_hbm, outx_hbm,
                 idxf_v, outx_v, xyz_v, buf0, buf1, buf2,
                 gsem0, gsem1, gsem2, ssem0, ssem1, ssem2)


_IDX_CACHE = None


def _sample_idx() -> np.ndarray:
    global _IDX_CACHE
    if _IDX_CACHE is None:
        try:
            dev_ctx = jax.default_device(jax.local_devices(backend="cpu")[0])
        except Exception:
            dev_ctx = contextlib.nullcontext()
        with jax.ensure_compile_time_eval(), dev_ctx:
            keys = jax.random.split(jax.random.key(42), B)
            perm = jax.vmap(lambda k: jax.random.permutation(k, N))(keys)
            _IDX_CACHE = np.asarray(perm[:, :NSAMPLE]).astype(np.int32)
    return _IDX_CACHE


def kernel(xyz, feature):
    idx = _sample_idx()

    idx_flat = idx + (np.arange(B, dtype=np.int32) * N)[:, None]
    idxf = jnp.asarray(idx_flat.reshape(_NW, _NCHUNK, _CHUNK))

    featf = feature.reshape(B * N, C)
    xyzt = jnp.transpose(xyz, (2, 0, 1))

    outf, outx = _sc_gather(featf, xyzt, idxf)

    feat_s = outf.reshape(B, NSAMPLE, C)
    xyz_s = jnp.transpose(outx, (1, 2, 0))
    return (xyz_s, feat_s)


# speedup vs baseline: 1.0071x; 1.0071x over previous
import contextlib
import functools

import jax
import jax.numpy as jnp
import numpy as np
from jax import lax
from jax.experimental import pallas as pl
from jax.experimental.pallas import tpu as pltpu
from jax.experimental.pallas import tpu_sc as plsc

B = 16
N = 16384
C = 256
NSAMPLE = 4096

_NC = 2
_NS = 16
_NW = _NC * _NS
_RPW = B * NSAMPLE // _NW
_CHUNK = 64
_NCHUNK = _RPW // _CHUNK


def _gather_body(feat_hbm, xyz_hbm, idxf_hbm,
                 outf_hbm, outx_hbm,
                 idxf_v, outx_v, xyz_v, buf0, buf1, buf2,
                 gsem0, gsem1, gsem2, ssem0, ssem1, ssem2):
    wid = lax.axis_index("s") * _NC + lax.axis_index("c")
    b = wid // 2
    h = wid % 2

    pltpu.sync_copy(idxf_hbm.at[wid], idxf_v)

    gsems = (gsem0, gsem1, gsem2)
    ssems = (ssem0, ssem1, ssem2)
    row_base = wid * _RPW
    bufs = (buf0, buf1, buf2)
    cps = [None, None, None]
    scs = [None, None, None]

    for k in range(2):
        cps[k] = pltpu.async_copy(
            feat_hbm.at[idxf_v.at[k]], bufs[k], gsems[k])

    for d in range(3):
        pltpu.sync_copy(xyz_hbm.at[d, pl.ds(b, 1)],
                        xyz_v.at[pl.ds(d, 1)])

    def _xyz_body(j, carry):
        r = j >> 2
        c = (j & 3) * 16
        ids = idxf_v[r, pl.ds(c, 16)] - b * N
        for d in range(3):
            dvec = jnp.full((16,), d, jnp.int32)
            outx_v[d, pl.ds(j * 16, 16)] = plsc.load_gather(
                xyz_v, [dvec, ids])
        return carry

    lax.fori_loop(0, _RPW // 16, _xyz_body, 0)
    for d in range(3):
        pltpu.sync_copy(outx_v.at[pl.ds(d, 1)],
                        outx_hbm.at[d, pl.ds(b, 1),
                                    pl.ds(h * _RPW, _RPW)])

    for i in range(_NCHUNK):
        cps[i % 3].wait()
        scs[i % 3] = pltpu.async_copy(
            bufs[i % 3],
            outf_hbm.at[pl.ds(row_base + i * _CHUNK, _CHUNK)],
            ssems[i % 3])
        nxt = i + 2
        if nxt < _NCHUNK:
            if i >= 1:
                scs[(i - 1) % 3].wait()
            cps[nxt % 3] = pltpu.async_copy(
                feat_hbm.at[idxf_v.at[nxt]], bufs[nxt % 3], gsems[nxt % 3])
    for i in range(_NCHUNK - 3, _NCHUNK):
        scs[i % 3].wait()


@functools.partial(
    pl.kernel,
    out_type=(
        jax.ShapeDtypeStruct((B * NSAMPLE, C), jnp.float32),
        jax.ShapeDtypeStruct((3, B, NSAMPLE), jnp.float32),
    ),
    mesh=plsc.VectorSubcoreMesh(core_axis_name="c", subcore_axis_name="s"),
    compiler_params=pltpu.CompilerParams(needs_layout_passes=False),
    scratch_types=[
        pltpu.VMEM((_NCHUNK, _CHUNK), jnp.int32),
        pltpu.VMEM((3, _RPW), jnp.float32),
        pltpu.VMEM((3, N), jnp.float32),
        pltpu.VMEM((_CHUNK, C), jnp.float32),
        pltpu.VMEM((_CHUNK, C), jnp.float32),
        pltpu.VMEM((_CHUNK, C), jnp.float32),
        pltpu.SemaphoreType.DMA,
        pltpu.SemaphoreType.DMA,
        pltpu.SemaphoreType.DMA,
        pltpu.SemaphoreType.DMA,
        pltpu.SemaphoreType.DMA,
        pltpu.SemaphoreType.DMA,
    ],
)
def _sc_gather(feat_hbm, xyz_hbm, idxf_hbm, outf_hbm, outx_hbm,
               idxf_v, outx_v, xyz_v, buf0, buf1, buf2,
               gsem0, gsem1, gsem2, ssem0, ssem1, ssem2):
    _gather_body(feat_hbm, xyz_hbm, idxf_hbm,
                 outf_hbm, outx_hbm,
                 idxf_v, outx_v, xyz_v, buf0, buf1, buf2,
                 gsem0, gsem1, gsem2, ssem0, ssem1, ssem2)


_IDX_CACHE = None


def _sample_idx() -> np.ndarray:
    global _IDX_CACHE
    if _IDX_CACHE is None:
        try:
            dev_ctx = jax.default_device(jax.local_devices(backend="cpu")[0])
        except Exception:
            dev_ctx = contextlib.nullcontext()
        with jax.ensure_compile_time_eval(), dev_ctx:
            keys = jax.random.split(jax.random.key(42), B)
            perm = jax.vmap(lambda k: jax.random.permutation(k, N))(keys)
            _IDX_CACHE = np.asarray(perm[:, :NSAMPLE]).astype(np.int32)
    return _IDX_CACHE


def kernel(xyz, feature):
    idx = _sample_idx()

    idx_flat = idx + (np.arange(B, dtype=np.int32) * N)[:, None]
    idxf = jnp.asarray(idx_flat.reshape(_NW, _NCHUNK, _CHUNK))

    featf = feature.reshape(B * N, C)
    xyzt = jnp.transpose(xyz, (2, 0, 1))

    outf, outx = _sc_gather(featf, xyzt, idxf)

    feat_s = outf.reshape(B, NSAMPLE, C)
    xyz_s = jnp.transpose(outx, (1, 2, 0))
    return (xyz_s, feat_s)


# speedup vs baseline: 1.0436x; 1.0362x over previous
import contextlib
import functools

import jax
import jax.numpy as jnp
import numpy as np
from jax import lax
from jax.experimental import pallas as pl
from jax.experimental.pallas import tpu as pltpu
from jax.experimental.pallas import tpu_sc as plsc

B = 16
N = 16384
C = 256
NSAMPLE = 4096

_NC = 2
_NS = 16
_NW = _NC * _NS
_RPW = B * NSAMPLE // _NW
_CHUNK = 64
_NCHUNK = _RPW // _CHUNK


def _gather_body(feat_hbm, xyz_hbm, idxf_hbm,
                 outf_hbm, outx_hbm,
                 idxf_v, outx_v, xyz_v, buf0, buf1, buf2,
                 gsem0, gsem1, gsem2, ssem0, ssem1, ssem2):
    wid = lax.axis_index("s") * _NC + lax.axis_index("c")
    b = wid // 2
    h = wid % 2

    pltpu.sync_copy(idxf_hbm.at[wid], idxf_v)

    gsems = (gsem0, gsem1, gsem2)
    ssems = (ssem0, ssem1, ssem2)
    row_base = wid * _RPW
    bufs = (buf0, buf1, buf2)
    cps = [None, None, None]
    scs = [None, None, None]

    for k in range(2):
        cps[k] = pltpu.async_copy(
            feat_hbm.at[idxf_v.at[k]], bufs[k], gsems[k])

    for d in range(3):
        pltpu.sync_copy(xyz_hbm.at[d, pl.ds(b, 1)],
                        xyz_v.at[pl.ds(d, 1)])

    def _xyz_body(j, carry):
        r = j >> 2
        c = (j & 3) * 16
        ids = idxf_v[r, pl.ds(c, 16)] - b * N
        for d in range(3):
            dvec = jnp.full((16,), d, jnp.int32)
            outx_v[d, pl.ds(j * 16, 16)] = plsc.load_gather(
                xyz_v, [dvec, ids])
        return carry

    lax.fori_loop(0, _RPW // 16, _xyz_body, 0)
    for d in range(3):
        pltpu.sync_copy(outx_v.at[pl.ds(d, 1)],
                        outx_hbm.at[d, pl.ds(b, 1),
                                    pl.ds(h * _RPW, _RPW)])

    def _wait_gather(i, bb):
        pltpu.make_async_copy(
            feat_hbm.at[idxf_v.at[i]], bufs[bb], gsems[bb]).wait()

    def _fire_gather(i, bb):
        pltpu.async_copy(feat_hbm.at[idxf_v.at[i]], bufs[bb], gsems[bb])

    def _fire_store(i, bb):
        pltpu.async_copy(
            bufs[bb], outf_hbm.at[pl.ds(row_base + i * _CHUNK, _CHUNK)],
            ssems[bb])

    def _wait_store(i, bb):
        pltpu.make_async_copy(
            bufs[bb], outf_hbm.at[pl.ds(row_base + i * _CHUNK, _CHUNK)],
            ssems[bb]).wait()

    _wait_gather(0, 0); _fire_store(0, 0); _fire_gather(2, 2)
    _wait_gather(1, 1); _fire_store(1, 1); _wait_store(0, 0); _fire_gather(3, 0)
    _wait_gather(2, 2); _fire_store(2, 2); _wait_store(1, 1); _fire_gather(4, 1)

    def _block(k, carry):
        for bb in range(3):
            i = k * 3 + bb
            _wait_gather(i, bb)
            _fire_store(i, bb)
            _wait_store(i - 1, (bb + 2) % 3)
            _fire_gather(i + 2, (bb + 2) % 3)
        return carry

    lax.fori_loop(1, _NCHUNK // 3, _block, 0)

    _wait_gather(30, 0); _fire_store(30, 0); _wait_store(29, 2)
    _wait_gather(31, 1); _fire_store(31, 1); _wait_store(30, 0)
    _wait_store(31, 1)


@functools.partial(
    pl.kernel,
    out_type=(
        jax.ShapeDtypeStruct((B * NSAMPLE, C), jnp.float32),
        jax.ShapeDtypeStruct((3, B, NSAMPLE), jnp.float32),
    ),
    mesh=plsc.VectorSubcoreMesh(core_axis_name="c", subcore_axis_name="s"),
    compiler_params=pltpu.CompilerParams(needs_layout_passes=False),
    scratch_types=[
        pltpu.VMEM((_NCHUNK, _CHUNK), jnp.int32),
        pltpu.VMEM((3, _RPW), jnp.float32),
        pltpu.VMEM((3, N), jnp.float32),
        pltpu.VMEM((_CHUNK, C), jnp.float32),
        pltpu.VMEM((_CHUNK, C), jnp.float32),
        pltpu.VMEM((_CHUNK, C), jnp.float32),
        pltpu.SemaphoreType.DMA,
        pltpu.SemaphoreType.DMA,
        pltpu.SemaphoreType.DMA,
        pltpu.SemaphoreType.DMA,
        pltpu.SemaphoreType.DMA,
        pltpu.SemaphoreType.DMA,
    ],
)
def _sc_gather(feat_hbm, xyz_hbm, idxf_hbm, outf_hbm, outx_hbm,
               idxf_v, outx_v, xyz_v, buf0, buf1, buf2,
               gsem0, gsem1, gsem2, ssem0, ssem1, ssem2):
    _gather_body(feat_hbm, xyz_hbm, idxf_hbm,
                 outf_hbm, outx_hbm,
                 idxf_v, outx_v, xyz_v, buf0, buf1, buf2,
                 gsem0, gsem1, gsem2, ssem0, ssem1, ssem2)


_IDX_CACHE = None


def _sample_idx() -> np.ndarray:
    global _IDX_CACHE
    if _IDX_CACHE is None:
        try:
            dev_ctx = jax.default_device(jax.local_devices(backend="cpu")[0])
        except Exception:
            dev_ctx = contextlib.nullcontext()
        with jax.ensure_compile_time_eval(), dev_ctx:
            keys = jax.random.split(jax.random.key(42), B)
            perm = jax.vmap(lambda k: jax.random.permutation(k, N))(keys)
            _IDX_CACHE = np.asarray(perm[:, :NSAMPLE]).astype(np.int32)
    return _IDX_CACHE


def kernel(xyz, feature):
    idx = _sample_idx()

    idx_flat = idx + (np.arange(B, dtype=np.int32) * N)[:, None]
    idxf = jnp.asarray(idx_flat.reshape(_NW, _NCHUNK, _CHUNK))

    featf = feature.reshape(B * N, C)
    xyzt = jnp.transpose(xyz, (2, 0, 1))

    outf, outx = _sc_gather(featf, xyzt, idxf)

    feat_s = outf.reshape(B, NSAMPLE, C)
    xyz_s = jnp.transpose(outx, (1, 2, 0))
    return (xyz_s, feat_s)
